# Initial kernel scaffold; baseline (speedup 1.0000x reference)
import jax
import jax.numpy as jnp
from jax.experimental import pallas as pl


def kernel(x, rms_weight, gate_w, gate_b, w1, w3, w2):
    raise NotImplementedError("write your pallas kernel here")


# speedup vs baseline: 2.9166x; 2.9166x over previous
import jax
import jax.numpy as jnp
from jax.experimental import pallas as pl
from jax.experimental.pallas import tpu as pltpu

_T = 512
_H = 768
_DFF = 768
_E = 64
_EPS = 1e-6


def _router_kernel(x_ref, rw_ref, gw_ref, gb_ref, tbf_ref, topi_ref, topw_ref):
    x = x_ref[...]
    var = jnp.mean(x * x, axis=1, keepdims=True)
    t = x * jax.lax.rsqrt(var + _EPS) * rw_ref[...]
    logits = jax.lax.dot_general(
        t, gw_ref[...], (((1,), (1,)), ((), ())),
        preferred_element_type=jnp.float32) + gb_ref[...]
    iota = jax.lax.broadcasted_iota(jnp.int32, (_T, _E), 1)
    m1 = jnp.max(logits, axis=1, keepdims=True)
    i1 = jnp.min(jnp.where(logits == m1, iota, _E), axis=1, keepdims=True)
    l2 = jnp.where(iota == i1, -jnp.inf, logits)
    m2 = jnp.max(l2, axis=1, keepdims=True)
    i2 = jnp.min(jnp.where(l2 == m2, iota, _E), axis=1, keepdims=True)
    a = jnp.exp(m2 - m1)
    denom = 1.0 + a
    topi_ref[:, 0:1] = i1
    topi_ref[:, 1:2] = i2
    topw_ref[:, 0:1] = 1.0 / denom
    topw_ref[:, 1:2] = a / denom
    tbf_ref[...] = t.astype(jnp.bfloat16)


def _moe_kernel(tbf_ref, topi_ref, topw_ref, x_ref, w1_ref, w3_ref, w2_ref,
                o_ref):
    e = pl.program_id(0)
    t = tbf_ref[...]
    g = jax.lax.dot_general(
        t, w1_ref[0].astype(jnp.bfloat16), (((1,), (1,)), ((), ())),
        preferred_element_type=jnp.float32)
    u = jax.lax.dot_general(
        t, w3_ref[0].astype(jnp.bfloat16), (((1,), (1,)), ((), ())),
        preferred_element_type=jnp.float32)
    h = (g * jax.lax.logistic(g)) * u
    y = jax.lax.dot_general(
        h.astype(jnp.bfloat16), w2_ref[0].astype(jnp.bfloat16),
        (((1,), (1,)), ((), ())), preferred_element_type=jnp.float32)
    topi = topi_ref[...]
    topw = topw_ref[...]
    coef = (jnp.where(topi[:, 0:1] == e, topw[:, 0:1], 0.0) +
            jnp.where(topi[:, 1:2] == e, topw[:, 1:2], 0.0))
    contrib = coef * y

    @pl.when(e == 0)
    def _():
        o_ref[...] = x_ref[...] + contrib

    @pl.when(e != 0)
    def _():
        o_ref[...] += contrib


def kernel(x, rms_weight, gate_w, gate_b, w1, w3, w2):
    tbf, topi, topw = pl.pallas_call(
        _router_kernel,
        out_shape=(
            jax.ShapeDtypeStruct((_T, _H), jnp.bfloat16),
            jax.ShapeDtypeStruct((_T, 2), jnp.int32),
            jax.ShapeDtypeStruct((_T, 2), jnp.float32),
        ),
    )(x, rms_weight.reshape(1, _H), gate_w, gate_b.reshape(1, _E))

    out = pl.pallas_call(
        _moe_kernel,
        grid=(_E,),
        in_specs=[
            pl.BlockSpec((_T, _H), lambda e: (0, 0)),
            pl.BlockSpec((_T, 2), lambda e: (0, 0)),
            pl.BlockSpec((_T, 2), lambda e: (0, 0)),
            pl.BlockSpec((_T, _H), lambda e: (0, 0)),
            pl.BlockSpec((1, _DFF, _H), lambda e: (e, 0, 0)),
            pl.BlockSpec((1, _DFF, _H), lambda e: (e, 0, 0)),
            pl.BlockSpec((1, _H, _DFF), lambda e: (e, 0, 0)),
        ],
        out_specs=pl.BlockSpec((_T, _H), lambda e: (0, 0)),
        out_shape=jax.ShapeDtypeStruct((_T, _H), jnp.float32),
        compiler_params=pltpu.CompilerParams(
            dimension_semantics=("arbitrary",)),
    )(tbf, topi, topw, x, w1, w3, w2)
    return out
